# Initial kernel scaffold; baseline (speedup 1.0000x reference)
import jax
import jax.numpy as jnp
from jax.experimental import pallas as pl


def kernel(node_features, edge_features, edge_index):
    raise NotImplementedError("write your pallas kernel here")


# speedup vs baseline: 4.2656x; 4.2656x over previous
import functools

import jax
import jax.numpy as jnp
from jax import lax
from jax.experimental import pallas as pl
from jax.experimental.pallas import tpu as pltpu
from jax.experimental.pallas import tpu_sc as plsc

_N_EDGES = 320000
_D = 128
_LANES = 16
_NC = 2
_NS = 16
_NW = _NC * _NS
_E_PER_W = _N_EDGES // _NW
_C = 80
_NCHUNK = _E_PER_W // _C


def _edge_hadamard(nf_hbm, src_hbm, dst_hbm, out_hbm,
                   idx_s, idx_d, rows_s, rows_d, sem):
    wid = lax.axis_index("s") * _NC + lax.axis_index("c")
    base = wid * _E_PER_W
    pltpu.sync_copy(src_hbm.at[pl.ds(base, _E_PER_W)], idx_s)
    pltpu.sync_copy(dst_hbm.at[pl.ds(base, _E_PER_W)], idx_d)

    def chunk(c, carry):
        off = c * _C
        cp1 = pltpu.async_copy(nf_hbm.at[idx_s.at[pl.ds(off, _C)]], rows_s, sem)
        cp2 = pltpu.async_copy(nf_hbm.at[idx_d.at[pl.ds(off, _C)]], rows_d, sem)
        cp1.wait()
        cp2.wait()

        def mul_row(i, carry2):
            for j in range(_D // _LANES):
                a = rows_s[i, pl.ds(j * _LANES, _LANES)]
                b = rows_d[i, pl.ds(j * _LANES, _LANES)]
                rows_s[i, pl.ds(j * _LANES, _LANES)] = a * b
            return carry2

        lax.fori_loop(0, _C, mul_row, 0)
        pltpu.sync_copy(rows_s, out_hbm.at[pl.ds(base + off, _C)])
        return carry

    lax.fori_loop(0, _NCHUNK, chunk, 0)


@jax.jit
def _run(node_features, src, dst):
    mesh = plsc.VectorSubcoreMesh(core_axis_name="c", subcore_axis_name="s")
    fn = functools.partial(
        pl.kernel,
        mesh=mesh,
        out_type=jax.ShapeDtypeStruct((_N_EDGES, _D), jnp.float32),
        scratch_types=[
            pltpu.VMEM((_E_PER_W,), jnp.int32),
            pltpu.VMEM((_E_PER_W,), jnp.int32),
            pltpu.VMEM((_C, _D), jnp.float32),
            pltpu.VMEM((_C, _D), jnp.float32),
            pltpu.SemaphoreType.DMA,
        ],
    )(_edge_hadamard)
    return fn(node_features, src, dst)


def kernel(node_features, edge_features, edge_index):
    del edge_features
    src = edge_index[0].astype(jnp.int32)
    dst = edge_index[1].astype(jnp.int32)
    return _run(node_features, src, dst)


# speedup vs baseline: 6.3677x; 1.4928x over previous
import functools

import jax
import jax.numpy as jnp
from jax import lax
from jax.experimental import pallas as pl
from jax.experimental.pallas import tpu as pltpu
from jax.experimental.pallas import tpu_sc as plsc

_N_EDGES = 320000
_D = 128
_LANES = 16
_NC = 2
_NS = 16
_NW = _NC * _NS
_E_PER_W = _N_EDGES // _NW
_C = 40
_NCHUNK = _E_PER_W // _C


def _edge_hadamard(nf_hbm, src_hbm, dst_hbm, out_hbm,
                   idx_s, idx_d, rows_s, rows_d, prod, g0, g1, o0, o1):
    gsem = (g0, g1)
    osem = (o0, o1)
    wid = lax.axis_index("s") * _NC + lax.axis_index("c")
    base = wid * _E_PER_W
    pltpu.sync_copy(src_hbm.at[pl.ds(base, _E_PER_W)], idx_s)
    pltpu.sync_copy(dst_hbm.at[pl.ds(base, _E_PER_W)], idx_d)

    def issue_gather(c, b):
        off = c * _C
        pltpu.async_copy(nf_hbm.at[idx_s.at[pl.ds(off, _C)]], rows_s.at[b], gsem[b])
        pltpu.async_copy(nf_hbm.at[idx_d.at[pl.ds(off, _C)]], rows_d.at[b], gsem[b])

    def wait_gather(b):
        pltpu.make_async_copy(nf_hbm.at[idx_s.at[pl.ds(0, _C)]],
                              rows_s.at[b], gsem[b]).wait()
        pltpu.make_async_copy(nf_hbm.at[idx_d.at[pl.ds(0, _C)]],
                              rows_d.at[b], gsem[b]).wait()

    def multiply(b):
        def mul_row(i, carry):
            for j in range(_D // _LANES):
                sl = pl.ds(j * _LANES, _LANES)
                prod[b, i, sl] = rows_s[b, i, sl] * rows_d[b, i, sl]
            return carry
        lax.fori_loop(0, _C, mul_row, 0)

    def issue_store(c, b):
        pltpu.async_copy(prod.at[b], out_hbm.at[pl.ds(base + c * _C, _C)], osem[b])

    def wait_store(b):
        pltpu.make_async_copy(prod.at[b],
                              out_hbm.at[pl.ds(base, _C)], osem[b]).wait()

    issue_gather(0, 0)
    issue_gather(1, 1)
    for b in range(2):
        wait_gather(b)
        multiply(b)
        issue_store(b, b)
        issue_gather(b + 2, b)

    def body(i, carry):
        for b in range(2):
            c = 2 * i + b
            wait_gather(b)
            wait_store(b)
            multiply(b)
            issue_store(c, b)
            issue_gather(c + 2, b)
        return carry

    lax.fori_loop(1, _NCHUNK // 2 - 1, body, 0)

    for b in range(2):
        c = _NCHUNK - 2 + b
        wait_gather(b)
        wait_store(b)
        multiply(b)
        issue_store(c, b)
    wait_store(0)
    wait_store(1)


@jax.jit
def _run(node_features, src, dst):
    fn = functools.partial(
        pl.kernel,
        mesh=plsc.VectorSubcoreMesh(core_axis_name="c", subcore_axis_name="s"),
        out_type=jax.ShapeDtypeStruct((_N_EDGES, _D), jnp.float32),
        scratch_types=[
            pltpu.VMEM((_E_PER_W,), jnp.int32),
            pltpu.VMEM((_E_PER_W,), jnp.int32),
            pltpu.VMEM((2, _C, _D), jnp.float32),
            pltpu.VMEM((2, _C, _D), jnp.float32),
            pltpu.VMEM((2, _C, _D), jnp.float32),
            pltpu.SemaphoreType.DMA,
            pltpu.SemaphoreType.DMA,
            pltpu.SemaphoreType.DMA,
            pltpu.SemaphoreType.DMA,
        ],
    )(_edge_hadamard)
    return fn(node_features, src, dst)


def kernel(node_features, edge_features, edge_index):
    del edge_features
    src = edge_index[0].astype(jnp.int32)
    dst = edge_index[1].astype(jnp.int32)
    return _run(node_features, src, dst)


# speedup vs baseline: 8.7755x; 1.3781x over previous
import functools

import jax
import jax.numpy as jnp
from jax import lax
from jax.experimental import pallas as pl
from jax.experimental.pallas import tpu as pltpu
from jax.experimental.pallas import tpu_sc as plsc

_N_EDGES = 320000
_D = 128
_LANES = 16
_NC = 2
_NS = 16
_NW = _NC * _NS
_E_PER_W = _N_EDGES // _NW
_C = 40
_NCHUNK = _E_PER_W // _C


_N_NODES = 10000
_ROWS_PER_TILE = 624


def _edge_hadamard(nf_hbm, src_hbm, dst_hbm, out_hbm,
                   table_sh, idx_s, idx_d, rows_s, rows_d, prod,
                   g0, g1, o0, o1):
    gsem = (g0, g1)
    osem = (o0, o1)
    sid = lax.axis_index("s")
    wid = sid * _NC + lax.axis_index("c")
    base = wid * _E_PER_W
    row0 = pl.multiple_of(sid * _ROWS_PER_TILE, 8)
    pltpu.sync_copy(nf_hbm.at[pl.ds(row0, _ROWS_PER_TILE)],
                    table_sh.at[pl.ds(row0, _ROWS_PER_TILE)])

    @pl.when(sid == _NS - 1)
    def _load_tail():
        tail0 = _NS * _ROWS_PER_TILE
        tail_n = _N_NODES - tail0
        pltpu.sync_copy(nf_hbm.at[pl.ds(tail0, tail_n)],
                        table_sh.at[pl.ds(tail0, tail_n)])
    pltpu.sync_copy(src_hbm.at[pl.ds(base, _E_PER_W)], idx_s)
    pltpu.sync_copy(dst_hbm.at[pl.ds(base, _E_PER_W)], idx_d)
    plsc.subcore_barrier()

    def issue_gather(c, b):
        off = c * _C
        pltpu.async_copy(table_sh.at[idx_s.at[pl.ds(off, _C)]], rows_s.at[b], gsem[b])
        pltpu.async_copy(table_sh.at[idx_d.at[pl.ds(off, _C)]], rows_d.at[b], gsem[b])

    def wait_gather(b):
        pltpu.make_async_copy(table_sh.at[idx_s.at[pl.ds(0, _C)]],
                              rows_s.at[b], gsem[b]).wait()
        pltpu.make_async_copy(table_sh.at[idx_d.at[pl.ds(0, _C)]],
                              rows_d.at[b], gsem[b]).wait()

    def multiply(b):
        def mul_row(i, carry):
            for j in range(_D // _LANES):
                sl = pl.ds(j * _LANES, _LANES)
                prod[b, i, sl] = rows_s[b, i, sl] * rows_d[b, i, sl]
            return carry
        lax.fori_loop(0, _C, mul_row, 0)

    def issue_store(c, b):
        pltpu.async_copy(prod.at[b], out_hbm.at[pl.ds(base + c * _C, _C)], osem[b])

    def wait_store(b):
        pltpu.make_async_copy(prod.at[b],
                              out_hbm.at[pl.ds(base, _C)], osem[b]).wait()

    issue_gather(0, 0)
    issue_gather(1, 1)
    for b in range(2):
        wait_gather(b)
        multiply(b)
        issue_store(b, b)
        issue_gather(b + 2, b)

    def body(i, carry):
        for b in range(2):
            c = 2 * i + b
            wait_gather(b)
            wait_store(b)
            multiply(b)
            issue_store(c, b)
            issue_gather(c + 2, b)
        return carry

    lax.fori_loop(1, _NCHUNK // 2 - 1, body, 0)

    for b in range(2):
        c = _NCHUNK - 2 + b
        wait_gather(b)
        wait_store(b)
        multiply(b)
        issue_store(c, b)
    wait_store(0)
    wait_store(1)


@jax.jit
def _run(node_features, src, dst):
    fn = functools.partial(
        pl.kernel,
        mesh=plsc.VectorSubcoreMesh(core_axis_name="c", subcore_axis_name="s"),
        out_type=jax.ShapeDtypeStruct((_N_EDGES, _D), jnp.float32),
        scratch_types=[
            pltpu.VMEM_SHARED((_N_NODES, _D), jnp.float32),
            pltpu.VMEM((_E_PER_W,), jnp.int32),
            pltpu.VMEM((_E_PER_W,), jnp.int32),
            pltpu.VMEM((2, _C, _D), jnp.float32),
            pltpu.VMEM((2, _C, _D), jnp.float32),
            pltpu.VMEM((2, _C, _D), jnp.float32),
            pltpu.SemaphoreType.DMA,
            pltpu.SemaphoreType.DMA,
            pltpu.SemaphoreType.DMA,
            pltpu.SemaphoreType.DMA,
        ],
    )(_edge_hadamard)
    return fn(node_features, src, dst)


def kernel(node_features, edge_features, edge_index):
    del edge_features
    src = edge_index[0].astype(jnp.int32)
    dst = edge_index[1].astype(jnp.int32)
    return _run(node_features, src, dst)
